# Initial kernel scaffold; baseline (speedup 1.0000x reference)
import jax
import jax.numpy as jnp
from jax.experimental import pallas as pl


def kernel(x1, edge_index1, batch1, x2, edge_index2, batch2, W, b, p):
    raise NotImplementedError("write your pallas kernel here")


# speedup vs baseline: 5.7222x; 5.7222x over previous
import dataclasses
import functools

import jax
import jax.numpy as jnp
import numpy as np
from jax import lax
from jax.experimental import pallas as pl
from jax.experimental.pallas import tpu as pltpu
from jax.experimental.pallas import tpu_sc as plsc

N = 10000
E = 160000
D = 256
BM = 400
BI = 1024
BJ = 1024

ROUNDS = [(10000, 8000, 10240), (8000, 6400, 8192), (6400, 5120, 7168)]

NW = 32
NS = 16
LANES = 16
EPAD = 163840
ES = EPAD // NW
ESC = 5392
PIECE = 1024
NCH = 32

@functools.lru_cache(maxsize=None)
def _sc_params():
    cp = pltpu.CompilerParams()
    if "needs_layout_passes" in pltpu.CompilerParams.__dataclass_fields__:
        cp = dataclasses.replace(cp, needs_layout_passes=False)
    return cp


@functools.lru_cache(maxsize=None)
def _mesh():
    return plsc.VectorSubcoreMesh(core_axis_name="c", subcore_axis_name="s",
                                  num_cores=2, num_subcores=16)


def _mm_body(x_ref, w_ref, o_ref):
    o_ref[...] = jnp.dot(x_ref[...], w_ref[...],
                         preferred_element_type=jnp.float32,
                         precision=lax.Precision.HIGHEST)


def _matmul(x, W, n):
    return pl.pallas_call(
        _mm_body,
        grid=(n // BM,),
        in_specs=[pl.BlockSpec((BM, D), lambda i: (i, 0)),
                  pl.BlockSpec((D, D), lambda i: (0, 0))],
        out_specs=pl.BlockSpec((BM, D), lambda i: (i, 0)),
        out_shape=jax.ShapeDtypeStruct((n, D), jnp.float32),
    )(x[:n], W)


def _epi_body(acc_ref, xw_ref, dinv_ref, b_ref, o_ref):
    dv = dinv_ref[...]
    o_ref[...] = (acc_ref[...] + xw_ref[...] * (dv * dv)) + b_ref[...]


def _epilogue(seg, xw, dinv, b, n):
    return pl.pallas_call(
        _epi_body,
        grid=(n // BM,),
        in_specs=[pl.BlockSpec((BM, D), lambda i: (i, 0)),
                  pl.BlockSpec((BM, D), lambda i: (i, 0)),
                  pl.BlockSpec((BM, 1), lambda i: (i, 0)),
                  pl.BlockSpec((1, D), lambda i: (0, 0))],
        out_specs=pl.BlockSpec((BM, D), lambda i: (i, 0)),
        out_shape=jax.ShapeDtypeStruct((n, D), jnp.float32),
    )(seg, xw, dinv.reshape(n, 1), b.reshape(1, D))


def _rank_body(si_ref, s_ref, rank_ref, *, P):
    i0 = pl.program_id(0) * BI
    si = jnp.broadcast_to(si_ref[...].reshape(BI, 1), (BI, BJ))

    def body(jb, acc):
        sj = jnp.broadcast_to(
            s_ref[pl.ds(jb * BJ, BJ)].reshape(1, BJ), (BI, BJ))

        def all_lt():
            return jnp.where(sj >= si, 1.0, 0.0)

        def all_gt():
            return jnp.where(sj > si, 1.0, 0.0)

        def mixed():
            jg = jb * BJ + lax.broadcasted_iota(jnp.int32, (BI, BJ), 1)
            ig = i0 + lax.broadcasted_iota(jnp.int32, (BI, BJ), 0)
            gt = jnp.where(sj > si, 1.0, 0.0)
            eq = jnp.where(sj == si, 1.0, 0.0)
            jlt = jnp.where(jg < ig, 1.0, 0.0)
            return gt + eq * jlt

        contrib = lax.cond(
            (jb + 1) * BJ <= i0, all_lt,
            lambda: lax.cond(jb * BJ >= i0 + BI, all_gt, mixed))
        return acc + jnp.sum(contrib, axis=1)

    rank_ref[...] = lax.fori_loop(
        0, P // BJ, body, jnp.zeros((BI,), jnp.float32)).astype(jnp.int32)


def _rank(score, n, P):
    sp = jnp.concatenate(
        [score, jnp.full((P - n,), -jnp.inf, jnp.float32)])
    return pl.pallas_call(
        functools.partial(_rank_body, P=P),
        grid=(P // BI,),
        in_specs=[pl.BlockSpec((BI,), lambda i: (i,)),
                  pl.BlockSpec((P,), lambda i: (0,))],
        out_specs=pl.BlockSpec((BI,), lambda i: (i,)),
        out_shape=jax.ShapeDtypeStruct((P,), jnp.int32),
    )(sp, sp)


def _ds(x, l):
    if isinstance(x, int):
        return pl.ds(x, l)
    return pl.ds(pl.multiple_of(x, 8), l)


def _div_magic(q, nmax):
    d = np.arange(nmax, dtype=np.int64)
    want = d // q
    for S in range(16, 31):
        M = (1 << S) // q + 1
        if M * (nmax - 1) < 2**31 and np.all((d * M) >> S == want):
            return int(M), int(S)
    raise ValueError(q)


def _np_pad(n):
    return ((n + 1 + 255) // 256) * 256


def _lane():
    return lax.iota(jnp.int32, LANES)


def _shr(x, bits):
    return lax.shift_right_logical(x, jnp.full((LANES,), bits, jnp.int32))


def _runs(vs):
    lane = _lane()
    nxt = jnp.take_along_axis(vs, jnp.minimum(lane + 1, 15), axis=0,
                              mode="promise_in_bounds")
    prv = jnp.take_along_axis(vs, jnp.maximum(lane - 1, 0), axis=0,
                              mode="promise_in_bounds")
    is_end = (vs != nxt) | (lane == 15)
    is_start = (vs != prv) | (lane == 0)
    rstart = plsc.cummax(jnp.where(is_start, lane, 0))
    return is_end, lane - rstart


def _make_part(n_new, has_remap, emit_next):
    Q = ((-(-n_new // NW)) + 7) // 8 * 8
    M, S = _div_magic(Q, 16384)
    NP = _np_pad(n_new)
    SP = NP // NS
    out_type = [
        jax.ShapeDtypeStruct((NW * 2 * ESC + 4096,), jnp.int32),
        jax.ShapeDtypeStruct((NW * 64,), jnp.int32),
        jax.ShapeDtypeStruct((2 * NP,), jnp.float32),
    ]
    if emit_next:
        out_type += [
            jax.ShapeDtypeStruct((EPAD,), jnp.int32),
            jax.ShapeDtypeStruct((EPAD,), jnp.int32),
            jax.ShapeDtypeStruct((EPAD,), jnp.float32),
        ]
    scratch = [
        pltpu.VMEM((ES,), jnp.int32),
        pltpu.VMEM((ES,), jnp.int32),
        pltpu.VMEM((ES,), jnp.float32),
        pltpu.VMEM((ES,), jnp.int32),
        pltpu.VMEM((ES,), jnp.int32),
        pltpu.VMEM((ES,), jnp.float32),
        pltpu.VMEM((ES,), jnp.int32),
        pltpu.VMEM((2 * ESC,), jnp.int32),
        pltpu.VMEM((NP,), jnp.float32),
        pltpu.VMEM((64,), jnp.int32),
        pltpu.VMEM((64,), jnp.int32),
        pltpu.VMEM((64,), jnp.int32),
        pltpu.VMEM((SP,), jnp.float32),
        pltpu.VMEM((SP,), jnp.float32),
        pltpu.VMEM_SHARED((NS * NP,), jnp.float32),
        pltpu.SemaphoreType.DMA,
    ]

    def body(*refs):
        if has_remap:
            src_h, dst_h, val_h, nix_h = refs[:4]
            rest = refs[4:]
        else:
            src_h, dst_h, val_h = refs[:3]
            nix_h = None
            rest = refs[3:]
        if emit_next:
            sgd_h, desc_h, degp_h, srcn_h, dstn_h, valn_h = rest[:6]
            rest = rest[6:]
        else:
            sgd_h, desc_h, degp_h = rest[:3]
            rest = rest[3:]
        (sin, din, vin, rsb, rdb, vnb, owb, sgl, degt, cnt, off, c2,
         stacc, sttmp, spd, sem) = rest

        c = lax.axis_index("c")
        s = lax.axis_index("s")
        w = c * NS + s
        base = w * ES
        lane = _lane()

        pltpu.sync_copy(src_h.at[_ds(base, ES)], sin)
        pltpu.sync_copy(dst_h.at[_ds(base, ES)], din)
        pltpu.sync_copy(val_h.at[_ds(base, ES)], vin)

        if has_remap:
            @pl.loop(0, ES // LANES)
            def _(j):
                sl = _ds(j * LANES, LANES)
                valid = vin[sl] > 0.0
                spread = (j * LANES + lane) & 4095
                sin[sl] = jnp.where(valid, sin[sl], spread)
                din[sl] = jnp.where(valid, din[sl], spread)

            @pl.loop(0, ES // 128)
            def _(ch):
                sl = _ds(ch * 128, 128)
                pltpu.async_copy(nix_h.at[sin.at[sl]], rsb.at[sl], sem).wait()
                pltpu.async_copy(nix_h.at[din.at[sl]], rdb.at[sl], sem).wait()

        @pl.loop(0, ES // LANES)
        def _(j):
            sl = _ds(j * LANES, LANES)
            v = vin[sl]
            if has_remap:
                rs = rsb[sl]
                rd = rdb[sl]
                ms = rs >= 0
                md = rd >= 0
                sp_ = jnp.where(ms, rs, 0)
                dp_ = jnp.where(md, rd, 0)
                vn = (v * jnp.where(ms, 1.0, 0.0)) * jnp.where(md, 1.0, 0.0)
            else:
                sp_, dp_, vn = sin[sl], din[sl], v
            ow = jnp.where(vn > 0.0, _shr(dp_ * M, S), NW)
            rsb[sl] = sp_
            rdb[sl] = dp_
            vnb[sl] = vn
            owb[sl] = ow

        @pl.loop(0, 64 // LANES)
        def _(j):
            cnt[_ds(j * LANES, LANES)] = jnp.zeros((LANES,), jnp.int32)

        @pl.loop(0, NP // LANES)
        def _(j):
            degt[_ds(j * LANES, LANES)] = jnp.zeros((LANES,), jnp.float32)

        @pl.loop(0, ES // LANES)
        def _(j):
            sl = _ds(j * LANES, LANES)
            key = owb[sl] * LANES + lane
            ks, _u = plsc.sort_key_val(key, key)
            ows = _shr(ks, 4)
            is_end, rnk = _runs(ows)
            old = plsc.load_gather(cnt, [ows], mask=is_end)
            plsc.store_scatter(cnt, [ows], old + rnk + 1, mask=is_end)
            dp_ = rdb[sl]
            db = jnp.where(vnb[sl] > 0.0, dp_, n_new)
            dk = db * LANES + lane
            dks, _u2 = plsc.sort_key_val(dk, dk)
            dbs = _shr(dks, 4)
            dend, drnk = _runs(dbs)
            dold = plsc.load_gather(degt, [dbs], mask=dend)
            plsc.store_scatter(
                degt, [dbs], dold + (drnk + 1).astype(jnp.float32), mask=dend)

        carry = jnp.int32(0)
        for j3 in range(3):
            sl3 = _ds(j3 * LANES, LANES)
            a8 = (cnt[sl3] + 7) & (-8)
            ex = plsc.cumsum(a8) - a8
            off[sl3] = ex + carry
            c2[sl3] = ex + carry
            carry = carry + jnp.sum(a8)
        off[_ds(48, LANES)] = jnp.zeros((LANES,), jnp.int32)
        c2[_ds(48, LANES)] = jnp.zeros((LANES,), jnp.int32)

        pltpu.sync_copy(off, desc_h.at[_ds(w * 64, 64)])

        @pl.loop(0, 2 * ESC // LANES)
        def _(j):
            sgl[_ds(j * LANES, LANES)] = jnp.where(
                (lane & 1) == 1, n_new, 0)

        @pl.loop(0, ES // LANES)
        def _(j):
            sl = _ds(j * LANES, LANES)
            key = owb[sl] * LANES + lane
            ks, _u = plsc.sort_key_val(key, key)
            ows = _shr(ks, 4)
            ol = ks & 15
            is_end, rnk = _runs(ows)
            bases = plsc.load_gather(c2, [ows])
            pos = bases + rnk
            plsc.store_scatter(c2, [ows], pos + 1, mask=is_end)
            gi = j * LANES + ol
            sv = plsc.load_gather(rsb, [gi])
            dv = plsc.load_gather(rdb, [gi])
            plsc.store_scatter(sgl, [pos * 2], sv)
            plsc.store_scatter(sgl, [pos * 2 + 1], dv)

        pltpu.sync_copy(sgl, sgd_h.at[_ds(w * 2 * ESC, 2 * ESC)])
        if emit_next:
            pltpu.sync_copy(rsb, srcn_h.at[_ds(base, ES)])
            pltpu.sync_copy(rdb, dstn_h.at[_ds(base, ES)])
            pltpu.sync_copy(vnb, valn_h.at[_ds(base, ES)])

        pltpu.sync_copy(degt, spd.at[_ds(s * NP, NP)])
        plsc.subcore_barrier()
        pltpu.sync_copy(spd.at[_ds(s * SP, SP)], stacc)

        @pl.loop(1, NS)
        def _(t):
            pltpu.sync_copy(spd.at[_ds(t * NP + s * SP, SP)], sttmp)

            @pl.loop(0, SP // LANES)
            def _(j2):
                sl2 = _ds(j2 * LANES, LANES)
                stacc[sl2] = stacc[sl2] + sttmp[sl2]

        pltpu.sync_copy(stacc, degp_h.at[_ds(c * NP + s * SP, SP)])

    return pl.kernel(body, out_type=out_type, mesh=_mesh(),
                     compiler_params=_sc_params(),
                     scratch_types=scratch)


def _make_seg(n):
    Q = ((-(-n // NW)) + 7) // 8 * 8
    QA = Q + 2
    out_type = jax.ShapeDtypeStruct((NW * Q, D), jnp.float32)
    scratch = [
        pltpu.VMEM((n,), jnp.float32),
        pltpu.VMEM((NW * 64,), jnp.int32),
        pltpu.VMEM((QA, D), jnp.float32),
        pltpu.VMEM((2 * PIECE,), jnp.int32),
        pltpu.VMEM((PIECE + LANES,), jnp.int32),
        pltpu.VMEM((PIECE + LANES,), jnp.int32),
        pltpu.VMEM((PIECE + LANES,), jnp.float32),
        pltpu.VMEM((NCH, D), jnp.float32),
        pltpu.VMEM((NCH, D), jnp.float32),
        pltpu.SemaphoreType.DMA,
        pltpu.SemaphoreType.DMA,
    ]

    def body(xw_h, dinv_h, sgd_h, desc_h, seg_h, dinvv, descv, acc,
             stage, sidx, dloc, coef, rows0, rows1, sem0, sem1):
        c = lax.axis_index("c")
        s = lax.axis_index("s")
        w = c * NS + s
        lane = _lane()
        pltpu.sync_copy(dinv_h, dinvv)
        pltpu.sync_copy(desc_h, descv)

        @pl.loop(0, QA)
        def _(r):
            for cc in range(D // LANES):
                acc[r, _ds(cc * LANES, LANES)] = jnp.zeros(
                    (LANES,), jnp.float32)

        @pl.loop(0, NW)
        def _(sc):
            didx = jnp.full((LANES,), sc * 64 + w, jnp.int32) + \
                jnp.minimum(lane, 1)
            dv16 = plsc.load_gather(descv, [didx])
            o0 = dv16[0]
            o1 = dv16[1]
            seg_cnt = o1 - o0
            segbase = sc * 2 * ESC + o0 * 2

            @pl.loop(0, pl.cdiv(seg_cnt, PIECE))
            def _(pz):
                pcnt = jnp.minimum(seg_cnt - pz * PIECE, PIECE)
                pltpu.sync_copy(
                    sgd_h.at[_ds(segbase + pz * 2 * PIECE, 2 * PIECE)],
                    stage)
                trips = pl.cdiv(pcnt, NCH)

                @pl.loop(0, trips * (NCH // LANES))
                def _(vi):
                    ebase = vi * LANES
                    idx = ebase + lane
                    m_in = idx < pcnt
                    sv = plsc.load_gather(stage, [idx * 2])
                    dv = plsc.load_gather(stage, [idx * 2 + 1])
                    sv = jnp.where(m_in, sv, 0)
                    dv = jnp.where(m_in, dv, n)
                    dvs = plsc.load_gather(dinvv, [sv])
                    dvd = plsc.load_gather(dinvv, [jnp.minimum(dv, n - 1)])
                    cf = jnp.where(m_in, dvs * dvd, 0.0)
                    dl = jnp.clip(dv - w * Q, 0, QA - 1)
                    sidx[_ds(ebase, LANES)] = sv
                    dloc[_ds(ebase, LANES)] = dl
                    coef[_ds(ebase, LANES)] = cf

                def fire(ch, rows, sem):
                    pltpu.async_copy(
                        xw_h.at[sidx.at[_ds(ch * NCH, NCH)]], rows, sem)

                def drain(ch, rows, sem):
                    pltpu.make_async_copy(
                        xw_h.at[sidx.at[_ds(ch * NCH, NCH)]],
                        rows, sem).wait()

                def process(ch, rows):
                    @pl.loop(0, NCH // LANES)
                    def _(g):
                        e0 = ch * NCH + g * LANES
                        cf16 = coef[_ds(e0, LANES)]
                        dl16 = dloc[_ds(e0, LANES)]
                        for r2 in range(LANES):
                            cf = cf16[r2]
                            dl = dl16[r2]
                            row = g * LANES + r2
                            for cc in range(D // LANES):
                                v = rows[row, _ds(cc * LANES, LANES)]
                                plsc.addupdate(
                                    acc.at[dl, _ds(cc * LANES, LANES)],
                                    v * cf)

                fire(0, rows0, sem0)

                @pl.loop(0, pl.cdiv(trips, 2))
                def _(h):
                    c0 = 2 * h

                    @pl.when(c0 + 1 < trips)
                    def _():
                        fire(c0 + 1, rows1, sem1)
                    drain(c0, rows0, sem0)
                    process(c0, rows0)

                    @pl.when(c0 + 1 < trips)
                    def _():
                        @pl.when(c0 + 2 < trips)
                        def _():
                            fire(c0 + 2, rows0, sem0)
                        drain(c0 + 1, rows1, sem1)
                        process(c0 + 1, rows1)

        qw = jnp.minimum(n - w * Q, Q)

        @pl.loop(0, qw // 8)
        def _(rb):
            pltpu.sync_copy(acc.at[_ds(rb * 8, 8), :],
                            seg_h.at[_ds(w * Q + rb * 8, 8), :])

    return pl.kernel(body, out_type=out_type, mesh=_mesh(),
                     compiler_params=_sc_params(),
                     scratch_types=scratch)


def _make_perm(n, k, P):
    QP = ((-(-n // NW)) + 63) // 64 * 64
    out_type = [jax.ShapeDtypeStruct((P,), jnp.int32),
                jax.ShapeDtypeStruct((n + 64,), jnp.int32)]
    scratch = [
        pltpu.VMEM((64,), jnp.int32),
        pltpu.VMEM((1, 64), jnp.int32),
        pltpu.VMEM((64,), jnp.int32),
        pltpu.VMEM((64,), jnp.int32),
        pltpu.SemaphoreType.DMA,
    ]

    def body(rank_h, permb_h, nix_h, rkv, ridx2d, vals, niv, sem):
        c = lax.axis_index("c")
        s = lax.axis_index("s")
        w = c * NS + s
        lane = _lane()
        base = w * QP
        cnt = jnp.clip(n - base, 0, QP)

        @pl.loop(0, pl.cdiv(cnt, 64))
        def _(ch):
            b0 = base + ch * 64
            pltpu.sync_copy(rank_h.at[_ds(b0, 64)], rkv)
            pltpu.sync_copy(rank_h.at[_ds(b0, 64)], ridx2d.at[0])
            for jj in range(4):
                sl = _ds(jj * LANES, LANES)
                rk = rkv[sl]
                vals[sl] = b0 + jj * LANES + lane
                niv[sl] = jnp.where(rk < k, rk, -1)
            pltpu.async_copy(vals, permb_h.at[ridx2d.at[0]], sem).wait()
            pltpu.sync_copy(niv, nix_h.at[_ds(b0, 64)])

    return pl.kernel(body, out_type=out_type, mesh=_mesh(),
                     compiler_params=_sc_params(),
                     scratch_types=scratch)


def _make_pool(n, k):
    QK = 256
    out_type = jax.ShapeDtypeStruct((k, D), jnp.float32)
    scratch = [
        pltpu.VMEM((64,), jnp.int32),
        pltpu.VMEM((n,), jnp.float32),
        pltpu.VMEM((64 + LANES,), jnp.float32),
        pltpu.VMEM((64, D), jnp.float32),
        pltpu.SemaphoreType.DMA,
    ]

    def body(out_h, t_h, permb_h, buf_h, pidx, tvv, tb, rows, sem):
        c = lax.axis_index("c")
        s = lax.axis_index("s")
        w = c * NS + s
        base = w * QK
        cnt = jnp.clip(k - base, 0, QK)
        pltpu.sync_copy(t_h, tvv)

        @pl.loop(0, cnt // 64)
        def _(ch):
            b0 = base + ch * 64
            pltpu.sync_copy(permb_h.at[_ds(b0, 64)], pidx)
            pltpu.async_copy(out_h.at[pidx], rows, sem).wait()
            for jj in range(4):
                sl = _ds(jj * LANES, LANES)
                tb[sl] = plsc.load_gather(tvv, [pidx[sl]])

            for jj2 in range(4):
                t16 = tb[_ds(jj2 * LANES, LANES)]
                for r2 in range(LANES):
                    tr = t16[r2]
                    row = jj2 * LANES + r2
                    for cc in range(D // LANES):
                        sl2 = _ds(cc * LANES, LANES)
                        rows[row, sl2] = rows[row, sl2] * tr

            pltpu.sync_copy(rows, buf_h.at[_ds(b0, 64), :])

    return pl.kernel(body, out_type=out_type, mesh=_mesh(),
                     compiler_params=_sc_params(),
                     scratch_types=scratch)


_make_part = functools.lru_cache(maxsize=None)(_make_part)
_make_seg = functools.lru_cache(maxsize=None)(_make_seg)
_make_perm = functools.lru_cache(maxsize=None)(_make_perm)
_make_pool = functools.lru_cache(maxsize=None)(_make_pool)


def _forward(x, edge_index, W, b, p):
    u = p / jnp.linalg.norm(p)
    pad = EPAD - E
    srce = jnp.concatenate([edge_index[0], jnp.zeros((pad,), jnp.int32)])
    dste = jnp.concatenate([edge_index[1], jnp.zeros((pad,), jnp.int32)])
    vale = jnp.concatenate(
        [jnp.ones((E,), jnp.float32), jnp.zeros((pad,), jnp.float32)])
    nix = None
    for r, (n, k, P) in enumerate(ROUNDS):
        if r == 0:
            sgd, desc, degp = _make_part(n, False, False)(srce, dste, vale)
        elif r == 1:
            sgd, desc, degp, srce, dste, vale = _make_part(n, True, True)(
                srce, dste, vale, nix)
        else:
            sgd, desc, degp = _make_part(n, True, False)(srce, dste, vale, nix)
        NPd = _np_pad(n)
        deg = degp[:n] + degp[NPd:NPd + n]
        dinv = lax.rsqrt(deg + 1.0)
        xw = _matmul(x, W, n)
        seg = _make_seg(n)(xw, dinv, sgd, desc)
        out = _epilogue(seg, xw, dinv, b, n)
        score = out @ u
        t = jnp.tanh(score)
        rank = _rank(score, n, P)
        permb, nix = _make_perm(n, k, P)(rank)
        x = _make_pool(n, k)(out, t, permb)
    return x


def kernel(x1, edge_index1, batch1, x2, edge_index2, batch2, W, b, p):
    out1 = _forward(x1, edge_index1, W, b, p)
    out2 = _forward(x2, edge_index2, W, b, p)
    return (out1, out2)
